# baseline (speedup 1.0000x reference)
import functools

import jax
import jax.numpy as jnp
from jax import lax
from jax.experimental import pallas as pl
from jax.experimental.pallas import tpu as pltpu
from jax.experimental.pallas import tpu_sc as plsc

_LANES = 16
_ROWS = 16


def _sc_add_pos(ar_flat, table, x):
    B, T, N, D = x.shape
    num_tiles = T
    zero_row = table.shape[0] - 1
    nvec = D // _LANES
    nch = N // _ROWS
    tail = N - nch * _ROWS

    mesh = plsc.VectorSubcoreMesh(core_axis_name="c", subcore_axis_name="s")

    @functools.partial(
        pl.kernel,
        mesh=mesh,
        compiler_params=pltpu.CompilerParams(needs_layout_passes=False),
        out_type=jax.ShapeDtypeStruct((B, T, N, D), jnp.float32),
        scratch_types=[
            pltpu.VMEM((_LANES,), jnp.int32),
            pltpu.VMEM((1,), jnp.int32),
            pltpu.VMEM((1, D), jnp.float32),
            pltpu.VMEM((2, _ROWS, D), jnp.float32),
            pltpu.VMEM((2, _ROWS, D), jnp.float32),
            pltpu.VMEM((tail, D), jnp.float32),
            pltpu.VMEM((tail, D), jnp.float32),
            pltpu.SemaphoreType.DMA,
            pltpu.SemaphoreType.DMA((2,)),
            pltpu.SemaphoreType.DMA((2,)),
            pltpu.SemaphoreType.DMA,
            pltpu.SemaphoreType.DMA,
        ],
    )
    def sc_kernel(ar_hbm, table_hbm, x_hbm, out_hbm, ar_v, idx_v, pos_v,
                  in_buf, out_buf, tin, tout, psem, isems, osems, tis, tos):
        wid = lax.axis_index("s") * 2 + lax.axis_index("c")
        b = wid // T
        t = wid % T

        pltpu.sync_copy(ar_hbm, ar_v)
        lane = lax.iota(jnp.int32, _LANES)
        k = lane + (wid // _LANES) * _LANES
        b2 = (k // num_tiles) * 2
        w = plsc.load_gather(ar_v, [b2])
        h = plsc.load_gather(ar_v, [b2 + 1])
        h_safe = jnp.maximum(h, 1)
        tt = k % num_tiles
        idx = jnp.where(
            tt < w * h,
            (tt // h_safe) * num_tiles + (tt % h_safe),
            zero_row,
        )
        plsc.store_scatter(idx_v, [lane * 0], idx, mask=lane == (wid % _LANES))
        pltpu.async_copy(table_hbm.at[idx_v], pos_v, psem).wait()

        def g(c, s):
            return pltpu.make_async_copy(
                x_hbm.at[b, t, pl.ds(c * _ROWS, _ROWS), :],
                in_buf.at[s],
                isems.at[s],
            )

        def sc(c, s):
            return pltpu.make_async_copy(
                out_buf.at[s],
                out_hbm.at[b, t, pl.ds(c * _ROWS, _ROWS), :],
                osems.at[s],
            )

        tail_g = pltpu.make_async_copy(
            x_hbm.at[b, t, pl.ds(nch * _ROWS, tail), :], tin, tis
        )
        tail_g.start()
        g(0, 0).start()
        g(1, 1).start()

        def add_rows(s):
            def rowfn(r, carry):
                for j in range(nvec):
                    sl = pl.ds(j * _LANES, _LANES)
                    out_buf[s, r, sl] = in_buf[s, r, sl] + pos_v[0, sl]
                return carry

            lax.fori_loop(0, _ROWS, rowfn, 0)

        for s in (0, 1):
            g(s, s).wait()
            add_rows(s)
            sc(s, s).start()
            g(s + 2, s).start()

        def outer(o, carry):
            for s in (0, 1):
                c = 2 * o + s
                g(c, s).wait()
                sc(c - 2, s).wait()
                add_rows(s)
                sc(c, s).start()
                g(c + 2, s).start()
            return carry

        lax.fori_loop(1, nch // 2 - 1, outer, 0)

        for s in (0, 1):
            c = nch - 2 + s
            g(c, s).wait()
            sc(c - 2, s).wait()
            add_rows(s)
            sc(c, s).start()

        tail_g.wait()
        for r in range(tail):
            for j in range(nvec):
                sl = pl.ds(j * _LANES, _LANES)
                tout[r, sl] = tin[r, sl] + pos_v[0, sl]
        tail_s = pltpu.make_async_copy(
            tout, out_hbm.at[b, t, pl.ds(nch * _ROWS, tail), :], tos
        )
        tail_s.start()

        sc(nch - 2, 0).wait()
        sc(nch - 1, 1).wait()
        tail_s.wait()

    return sc_kernel(ar_flat, table, x)


def kernel(x, ar, embedding):
    B, T, N, D = x.shape
    nt = embedding.shape[0]
    ar_flat = ar.astype(jnp.int32).reshape(-1)
    table = jnp.concatenate(
        [embedding.reshape(nt * nt, D), jnp.zeros((1, D), embedding.dtype)],
        axis=0,
    )
    return _sc_add_pos(ar_flat, table, x)


# speedup vs baseline: 1.6328x; 1.6328x over previous
import functools

import jax
import jax.numpy as jnp
from jax import lax
from jax.experimental import pallas as pl
from jax.experimental.pallas import tpu as pltpu
from jax.experimental.pallas import tpu_sc as plsc

_LANES = 16


def _sc_gather_pos(ar_flat, table):
    n_rows_out = 32
    width = table.shape[1]
    num_tiles = 4
    zero_row = table.shape[0] - 1

    mesh = plsc.VectorSubcoreMesh(core_axis_name="c", subcore_axis_name="s")

    @functools.partial(
        pl.kernel,
        mesh=mesh,
        compiler_params=pltpu.CompilerParams(needs_layout_passes=False),
        out_type=jax.ShapeDtypeStruct((n_rows_out, width), jnp.float32),
        scratch_types=[
            pltpu.VMEM((_LANES,), jnp.int32),
            pltpu.VMEM((1,), jnp.int32),
            pltpu.VMEM((1, width), jnp.float32),
            pltpu.SemaphoreType.DMA,
        ],
    )
    def sc_kernel(ar_hbm, table_hbm, out_hbm, ar_v, idx_v, row_v, sem):
        wid = lax.axis_index("s") * 2 + lax.axis_index("c")
        pltpu.sync_copy(ar_hbm, ar_v)
        lane = lax.iota(jnp.int32, _LANES)
        k = lane + (wid // _LANES) * _LANES
        b2 = (k // num_tiles) * 2
        w = plsc.load_gather(ar_v, [b2])
        h = plsc.load_gather(ar_v, [b2 + 1])
        h_safe = jnp.maximum(h, 1)
        t = k % num_tiles
        idx = jnp.where(
            t < w * h,
            (t // h_safe) * num_tiles + (t % h_safe),
            zero_row,
        )
        plsc.store_scatter(idx_v, [lane * 0], idx, mask=lane == (wid % _LANES))
        pltpu.async_copy(table_hbm.at[idx_v], row_v, sem).wait()
        pltpu.sync_copy(row_v, out_hbm.at[pl.ds(wid, 1)])

    return sc_kernel(ar_flat, table)


def _tc_add(x, pos):
    B, T, N, D = x.shape
    NB = 128
    CPS = N // NB
    TAIL = N - CPS * NB
    NCHUNK = B * T * CPS
    NBUF = 32

    def body(x_hbm, pos_v, out_hbm, in_buf, out_buf, tail_in, tail_out,
             in_sems, out_sems, tail_in_sem, tail_out_sem):
        def in_copy(i, s):
            bt = i // CPS
            r = (i % CPS) * NB
            return pltpu.make_async_copy(
                x_hbm.at[bt // T, bt % T, pl.ds(r, NB), :],
                in_buf.at[s],
                in_sems.at[s],
            )

        def out_copy(i, s):
            bt = i // CPS
            r = (i % CPS) * NB
            return pltpu.make_async_copy(
                out_buf.at[s],
                out_hbm.at[bt // T, bt % T, pl.ds(r, NB), :],
                out_sems.at[s],
            )

        tail_in_cp = pltpu.make_async_copy(
            x_hbm.at[:, :, pl.ds(CPS * NB, TAIL), :], tail_in, tail_in_sem
        )

        tail_in_cp.start()
        for s in range(NBUF):
            in_copy(s, s).start()

        def step(i, carry):
            s = i % NBUF
            bt = i // CPS
            in_copy(i, s).wait()

            @pl.when(i >= NBUF)
            def _():
                out_copy(i - NBUF, s).wait()

            out_buf[s] = in_buf[s] + pos_v[bt // T, bt % T]
            out_copy(i, s).start()

            @pl.when(i + NBUF < NCHUNK)
            def _():
                in_copy(i + NBUF, s).start()

            return carry

        lax.fori_loop(0, NCHUNK, step, 0)

        tail_in_cp.wait()
        tail_out[...] = tail_in[...] + pos_v[...]
        tail_cp = pltpu.make_async_copy(
            tail_out, out_hbm.at[:, :, pl.ds(CPS * NB, TAIL), :], tail_out_sem
        )
        tail_cp.start()

        for s in range(NBUF):
            out_copy(NCHUNK - NBUF + s, s).wait()
        tail_cp.wait()

    return pl.pallas_call(
        body,
        in_specs=[
            pl.BlockSpec(memory_space=pl.ANY),
            pl.BlockSpec(memory_space=pltpu.MemorySpace.VMEM),
        ],
        out_specs=pl.BlockSpec(memory_space=pl.ANY),
        out_shape=jax.ShapeDtypeStruct(x.shape, x.dtype),
        scratch_shapes=[
            pltpu.VMEM((NBUF, NB, D), x.dtype),
            pltpu.VMEM((NBUF, NB, D), x.dtype),
            pltpu.VMEM((B, T, TAIL, D), x.dtype),
            pltpu.VMEM((B, T, TAIL, D), x.dtype),
            pltpu.SemaphoreType.DMA((NBUF,)),
            pltpu.SemaphoreType.DMA((NBUF,)),
            pltpu.SemaphoreType.DMA,
            pltpu.SemaphoreType.DMA,
        ],
        compiler_params=pltpu.CompilerParams(
            vmem_limit_bytes=56 * 1024 * 1024,
        ),
    )(x, pos.reshape(B, T, 1, D))


def kernel(x, ar, embedding):
    B, T, N, D = x.shape
    nt = embedding.shape[0]
    ar_flat = ar.astype(jnp.int32).reshape(-1)
    table = jnp.concatenate(
        [embedding.reshape(nt * nt, D), jnp.zeros((1, D), embedding.dtype)],
        axis=0,
    )
    pos = _sc_gather_pos(ar_flat, table)
    return _tc_add(x, pos)
